# baseline (speedup 1.0000x reference)
import functools

import jax
import jax.numpy as jnp
from jax import lax
from jax.experimental import pallas as pl
from jax.experimental.pallas import tpu as pltpu
from jax.experimental.pallas import tpu_sc as plsc


def _make_gather_t(D, V, B):
    info = plsc.get_sparse_core_info()
    NC, NS = info.num_cores, info.num_subcores
    NW = NC * NS
    assert B % NW == 0
    b_per_w = B // NW
    K = 8
    M = 5
    NB = b_per_w // K
    mesh = plsc.VectorSubcoreMesh(core_axis_name="c", subcore_axis_name="s")

    @functools.partial(
        pl.kernel,
        mesh=mesh,
        compiler_params=pltpu.CompilerParams(needs_layout_passes=False),
        out_type=jax.ShapeDtypeStruct((D, B), jnp.float32),
        scratch_types=[
            pltpu.VMEM((b_per_w + 16,), jnp.int32),
            pltpu.VMEM((M, K, D, 128), jnp.float32),
            pltpu.VMEM((D, b_per_w), jnp.float32),
        ]
        + [pltpu.SemaphoreType.DMA] * M,
    )
    def gather_kernel(table_hbm, idx_hbm, out_hbm, idx_v, win_v, obuf_v, *sems):
        wid = lax.axis_index("s") * NC + lax.axis_index("c")
        base = wid * b_per_w
        pltpu.sync_copy(idx_hbm.at[pl.ds(base, b_per_w)], idx_v.at[pl.ds(0, b_per_w)])
        rows0 = lax.iota(jnp.int32, 16)

        def fire(bidx, buf):
            v = idx_v[pl.ds(bidx * K, 16)]
            for m in range(K):
                off = pl.multiple_of((v[m] >> 7) * 128, 128)
                pltpu.async_copy(
                    table_hbm.at[:, pl.ds(off, 128)],
                    win_v.at[buf, m],
                    sems[buf],
                )

        def drain(buf):
            for m in range(K):
                pltpu.make_async_copy(
                    table_hbm.at[:, pl.ds(0, 128)], win_v.at[buf, m], sems[buf]
                ).wait()

        def extract(bidx, buf):
            v = idx_v[pl.ds(bidx * K, 16)]
            bsel = jnp.full((16,), buf, jnp.int32)
            for m in range(K):
                lane = jnp.full((16,), v[m] & 127, jnp.int32)
                col = jnp.full((16,), bidx * K + m, jnp.int32)
                val = plsc.load_gather(
                    win_v, [bsel, jnp.full((16,), m, jnp.int32), rows0, lane]
                )
                plsc.store_scatter(obuf_v, [rows0, col], val)

        for t in range(M - 1):
            fire(t, t)

        n_loop = ((NB - (M - 1)) // M) * M

        def body(q, carry):
            for r in range(M):
                t = M * q + r
                fire(t + (M - 1), (r + (M - 1)) % M)
                drain(r)
                extract(t, r)
            return carry

        lax.fori_loop(0, n_loop // M, body, 0)
        for t in range(n_loop, NB):
            if t + (M - 1) < NB:
                fire(t + (M - 1), (t + (M - 1)) % M)
            drain(t % M)
            extract(t, t % M)
        pltpu.sync_copy(obuf_v, out_hbm.at[:, pl.ds(base, b_per_w)])

    return gather_kernel


def kernel(stack_code, emb_table):
    B = stack_code.shape[0]
    V, D = emb_table.shape
    gather_t = _make_gather_t(D, V, B)
    out_t = gather_t(emb_table.T, stack_code.astype(jnp.int32))
    return out_t.T
